# baseline (speedup 1.0000x reference)
import functools

import jax
import jax.numpy as jnp
from jax import lax
from jax.experimental import pallas as pl
from jax.experimental.pallas import tpu as pltpu
from jax.experimental.pallas import tpu_sc as plsc

_D = 64
_B = 16384
_LAMBDA = 0.001
_NC, _NS, _L = 2, 16, 16
_NW = _NC * _NS
_BPW = _B // _NW
_CHUNK = 128
_NCH = _BPW // _CHUNK
_SCALE = _LAMBDA / _B

_mesh = plsc.VectorSubcoreMesh(core_axis_name="c", subcore_axis_name="s")


@functools.partial(
    pl.kernel,
    mesh=_mesh,
    out_type=jax.ShapeDtypeStruct((_NW, _L), jnp.float32),
    compiler_params=pltpu.CompilerParams(use_tc_tiling_on_sc=False),
    scratch_types=[
        pltpu.VMEM((_NCH, _CHUNK), jnp.int32),
        pltpu.VMEM((_NCH, _CHUNK, _D), jnp.float32),
        pltpu.VMEM((_NCH, _CHUNK, _D), jnp.float32),
        pltpu.VMEM((_L,), jnp.float32),
        pltpu.SemaphoreType.DMA,
    ],
)
def _center_loss_sc(feat_hbm, lab_hbm, cent_hbm, out_hbm,
                    lab_v, rows_v, feat_v, out_v, sem):
    wid = lax.axis_index("s") * _NC + lax.axis_index("c")
    pltpu.sync_copy(lab_hbm.at[wid], lab_v)
    feat_cp = pltpu.async_copy(feat_hbm.at[wid], feat_v, sem)
    gather_cps = [
        pltpu.async_copy(cent_hbm.at[lab_v.at[ch]], rows_v.at[ch], sem)
        for ch in range(_NCH)
    ]
    feat_cp.wait()
    for cp in gather_cps:
        cp.wait()

    acc = jnp.zeros((_L,), jnp.float32)
    for ch in range(_NCH):
        def body(i, a, ch=ch):
            for d in range(_D // _L):
                f = feat_v[ch, i, pl.ds(d * _L, _L)]
                c = rows_v[ch, i, pl.ds(d * _L, _L)]
                df = f - c
                a = a + df * df
            return a
        acc = lax.fori_loop(0, _CHUNK, body, acc)

    out_v[...] = acc * _SCALE
    pltpu.sync_copy(out_v, out_hbm.at[wid])


def kernel(features, labels, centers):
    feat = features.reshape(_NW, _NCH, _CHUNK, _D)
    lab = labels.astype(jnp.int32).reshape(_NW, _NCH, _CHUNK)
    partials = _center_loss_sc(feat, lab, centers)
    return jnp.sum(partials)
